# baseline (speedup 1.0000x reference)
import jax
import jax.numpy as jnp
from jax import lax
from jax.experimental import pallas as pl
from jax.experimental.pallas import tpu as pltpu
from jax.experimental.pallas import tpu_sc as plsc

_NUM_EMB_PER_CH = 256
_IN_CHANNELS = 192
_MIN_SCALE = 2.5
_MAX_SCALE = 3.5

_H = 224
_W = 224
_BH = 16
_WVECS = _W // 16
_BLOCKS_PER_PLANE = _H // _BH

_NC = 2
_NS = 16
_NW = _NC * _NS
_PLANES_PER_W = 768 // _NW
_BLOCKS_PER_W = _PLANES_PER_W * _BLOCKS_PER_PLANE


def _body(x_hbm, tab_hbm, tabs_hbm, scale_hbm, out_hbm,
          tab_v, tabs_v, scale_v, xin0, xin1, xout0, xout1,
          isem0, isem1, osem0, osem1):
    wid = lax.axis_index("s") * _NC + lax.axis_index("c")

    pltpu.sync_copy(tab_hbm, tab_v)
    pltpu.sync_copy(tabs_hbm, tabs_v)
    pltpu.sync_copy(scale_hbm, scale_v)

    def block_loc(k):
        plane = wid * _PLANES_PER_W + k // _BLOCKS_PER_PLANE
        return plane, (k % _BLOCKS_PER_PLANE) * _BH

    def start_in(k, buf, sem):
        plane, h0 = block_loc(k)
        pltpu.async_copy(x_hbm.at[plane, pl.ds(h0, _BH)], buf, sem)

    start_in(0, xin0, isem0)
    start_in(1, xin1, isem1)

    bufs = ((xin0, isem0, xout0, osem0), (xin1, isem1, xout1, osem1))

    def step(j, _):
        for b, (xin, isem, xout, osem) in enumerate(bufs):
            k = 2 * j + b
            plane, h0 = block_loc(k)
            ch = plane % _IN_CHANNELS

            pltpu.make_async_copy(
                x_hbm.at[plane, pl.ds(h0, _BH)], xin, isem).wait()

            @pl.when(j > 0)
            def _wait_out():
                pltpu.make_async_copy(
                    xout, out_hbm.at[plane, pl.ds(h0, _BH)], osem).wait()

            ch_vec = jnp.full((16,), ch, dtype=jnp.int32)
            sv = plsc.load_gather(scale_v, [ch_vec])
            sv = jnp.minimum(jnp.maximum(sv, _MIN_SCALE), _MAX_SCALE)
            nsv = -sv
            offv = jnp.full(
                (16,), ch * _NUM_EMB_PER_CH, dtype=jnp.int32
            ).astype(jnp.float32)

            @plsc.parallel_loop(0, _BH, unroll=2)
            def _row(h):
                for w in range(_WVECS):
                    xv = xin[h, pl.ds(w * 16, 16)]
                    t = jnp.minimum(jnp.maximum(xv, nsv), sv)
                    o4 = (t / sv + 1.0) * 127.5 + offv
                    li = o4.astype(jnp.int32)
                    lof = li.astype(jnp.float32)
                    d = o4 - lof
                    lov = plsc.load_gather(tab_v, [li])
                    upv = plsc.load_gather(tabs_v, [li])
                    blend = d * lov + (1.0 - d) * upv
                    xout[h, pl.ds(w * 16, 16)] = jnp.where(d > 0.0, blend, lov)

            pltpu.async_copy(xout, out_hbm.at[plane, pl.ds(h0, _BH)], osem)

            @pl.when(j < _BLOCKS_PER_W // 2 - 1)
            def _refill():
                start_in(k + 2, xin, isem)
        return 0

    lax.fori_loop(0, _BLOCKS_PER_W // 2, step, 0)

    p0, h0 = block_loc(_BLOCKS_PER_W - 2)
    p1, h1 = block_loc(_BLOCKS_PER_W - 1)
    pltpu.make_async_copy(
        xout0, out_hbm.at[p0, pl.ds(h0, _BH)], osem0).wait()
    pltpu.make_async_copy(
        xout1, out_hbm.at[p1, pl.ds(h1, _BH)], osem1).wait()


@jax.jit
def kernel(x, value_embeddings, scale):
    tab = value_embeddings.reshape(-1)
    tabshift = jnp.concatenate([tab[1:], tab[-1:]])
    sc = scale.reshape(-1)
    x3 = x.reshape(768, _H, _W)

    mesh = plsc.VectorSubcoreMesh(
        core_axis_name="c", subcore_axis_name="s", num_cores=_NC,
        num_subcores=_NS,
    )
    run = pl.kernel(
        _body,
        out_type=jax.ShapeDtypeStruct((768, _H, _W), jnp.float32),
        mesh=mesh,
        scratch_types=[
            pltpu.VMEM((tab.shape[0],), jnp.float32),
            pltpu.VMEM((tab.shape[0],), jnp.float32),
            pltpu.VMEM((_IN_CHANNELS,), jnp.float32),
            pltpu.VMEM((_BH, _W), jnp.float32),
            pltpu.VMEM((_BH, _W), jnp.float32),
            pltpu.VMEM((_BH, _W), jnp.float32),
            pltpu.VMEM((_BH, _W), jnp.float32),
            pltpu.SemaphoreType.DMA,
            pltpu.SemaphoreType.DMA,
            pltpu.SemaphoreType.DMA,
            pltpu.SemaphoreType.DMA,
        ],
        compiler_params=pltpu.CompilerParams(
            needs_layout_passes=False,
            use_tc_tiling_on_sc=True,
        ),
    )
    out = run(x3, tab, tabshift, sc)
    return out.reshape(x.shape)


# speedup vs baseline: 1.1150x; 1.1150x over previous
import jax
import jax.numpy as jnp
from jax import lax
from jax.experimental import pallas as pl
from jax.experimental.pallas import tpu as pltpu
from jax.experimental.pallas import tpu_sc as plsc

_NUM_EMB_PER_CH = 256
_IN_CHANNELS = 192
_MIN_SCALE = 2.5
_MAX_SCALE = 3.5

_H = 224
_W = 224
_BH = 56
_WVECS = _W // 16

_NC = 2
_NS = 16
_NW = _NC * _NS
_PLANES_PER_W = 768 // _NW
_BLOCKS_PER_W = _PLANES_PER_W * 4


def _body(x_hbm, tab_hbm, scale_hbm, out_hbm,
          tab_v, scale_v, xin0, xin1, xout0, xout1,
          isem0, isem1, osem0, osem1):
    wid = lax.axis_index("s") * _NC + lax.axis_index("c")

    pltpu.sync_copy(tab_hbm, tab_v)
    pltpu.sync_copy(scale_hbm, scale_v)

    def block_loc(k):
        plane = wid * _PLANES_PER_W + k // 4
        return plane, (k % 4) * _BH

    def start_in(k, buf, sem):
        plane, h0 = block_loc(k)
        pltpu.async_copy(x_hbm.at[plane, pl.ds(h0, _BH)], buf, sem)

    start_in(0, xin0, isem0)
    start_in(1, xin1, isem1)

    bufs = ((xin0, isem0, xout0, osem0), (xin1, isem1, xout1, osem1))

    def step(j, _):
        for b, (xin, isem, xout, osem) in enumerate(bufs):
            k = 2 * j + b
            plane, h0 = block_loc(k)
            ch = plane % _IN_CHANNELS

            pltpu.make_async_copy(
                x_hbm.at[plane, pl.ds(h0, _BH)], xin, isem).wait()

            @pl.when(j > 0)
            def _wait_out():
                pltpu.make_async_copy(
                    xout, out_hbm.at[plane, pl.ds(h0, _BH)], osem).wait()

            ch_vec = jnp.full((16,), ch, dtype=jnp.int32)
            sv = plsc.load_gather(scale_v, [ch_vec])
            sv = jnp.minimum(jnp.maximum(sv, _MIN_SCALE), _MAX_SCALE)
            nsv = -sv
            offv = jnp.full(
                (16,), ch * _NUM_EMB_PER_CH, dtype=jnp.int32
            ).astype(jnp.float32)

            @plsc.parallel_loop(0, _BH, unroll=2)
            def _row(h):
                for w in range(_WVECS):
                    xv = xin[h, pl.ds(w * 16, 16)]
                    t = jnp.minimum(jnp.maximum(xv, nsv), sv)
                    o4 = (t / sv + 1.0) * 127.5 + offv
                    li = o4.astype(jnp.int32)
                    lof = li.astype(jnp.float32)
                    d = o4 - lof
                    ui = li + (d > 0.0).astype(jnp.int32)
                    lov = plsc.load_gather(tab_v, [li])
                    upv = plsc.load_gather(tab_v, [ui])
                    xout[h, pl.ds(w * 16, 16)] = d * lov + (1.0 - d) * upv

            pltpu.async_copy(xout, out_hbm.at[plane, pl.ds(h0, _BH)], osem)

            @pl.when(j < _BLOCKS_PER_W // 2 - 1)
            def _refill():
                start_in(k + 2, xin, isem)
        return 0

    lax.fori_loop(0, _BLOCKS_PER_W // 2, step, 0)

    p0, h0 = block_loc(_BLOCKS_PER_W - 2)
    p1, h1 = block_loc(_BLOCKS_PER_W - 1)
    pltpu.make_async_copy(
        xout0, out_hbm.at[p0, pl.ds(h0, _BH)], osem0).wait()
    pltpu.make_async_copy(
        xout1, out_hbm.at[p1, pl.ds(h1, _BH)], osem1).wait()


@jax.jit
def kernel(x, value_embeddings, scale):
    tab = value_embeddings.reshape(-1)
    sc = scale.reshape(-1)
    x3 = x.reshape(768, _H, _W)

    mesh = plsc.VectorSubcoreMesh(
        core_axis_name="c", subcore_axis_name="s", num_cores=_NC,
        num_subcores=_NS,
    )
    run = pl.kernel(
        _body,
        out_type=jax.ShapeDtypeStruct((768, _H, _W), jnp.float32),
        mesh=mesh,
        scratch_types=[
            pltpu.VMEM((tab.shape[0],), jnp.float32),
            pltpu.VMEM((_IN_CHANNELS,), jnp.float32),
            pltpu.VMEM((_BH, _W), jnp.float32),
            pltpu.VMEM((_BH, _W), jnp.float32),
            pltpu.VMEM((_BH, _W), jnp.float32),
            pltpu.VMEM((_BH, _W), jnp.float32),
            pltpu.SemaphoreType.DMA,
            pltpu.SemaphoreType.DMA,
            pltpu.SemaphoreType.DMA,
            pltpu.SemaphoreType.DMA,
        ],
        compiler_params=pltpu.CompilerParams(
            needs_layout_passes=False,
            use_tc_tiling_on_sc=True,
        ),
    )
    out = run(x3, tab, sc)
    return out.reshape(x.shape)
